# baseline (speedup 1.0000x reference)
import functools

import jax
import jax.numpy as jnp
from jax import lax
from jax.experimental import pallas as pl
from jax.experimental.pallas import tpu as pltpu
from jax.experimental.pallas import tpu_sc as plsc

N = 10000
E = 320000
DIN = 128
H0 = 128
H1 = 64
G = 256

BLK = 400
NBLK = N // BLK

NC = 2
NS = 16
NW = NC * NS
EW = E // NW
K = 128
EWP = 10240
NCHUNK = EWP // K
HK = K // 2
RPT = 640
ZROWS = 32

_mesh = plsc.VectorSubcoreMesh(core_axis_name="c", subcore_axis_name="s")


NPAD = 10240


def _deg_body(dst_hbm, w_hbm, out_hbm, dst2_v, w2_v, zb_v, dacc):
    cid = lax.axis_index("c")
    sid = lax.axis_index("s")
    wid = sid * NC + cid

    pltpu.sync_copy(dst_hbm.at[wid], dst2_v)
    pltpu.sync_copy(w_hbm.at[wid], w2_v)

    def zero_row(r, carry):
        zb_v[pl.ds(r * 16, 16)] = jnp.zeros((16,), jnp.float32)
        return carry
    lax.fori_loop(0, 40, zero_row, 0)

    off = pl.multiple_of(sid * 640, 128)
    pltpu.sync_copy(zb_v, dacc.at[pl.ds(off, 640)])

    plsc.subcore_barrier()

    def chunk(c, carry):
        pltpu.sync_copy(w2_v.at[c], dacc.at[dst2_v.at[c]], add=True)
        return carry
    lax.fori_loop(0, NCHUNK, chunk, 0)

    plsc.subcore_barrier()

    pltpu.sync_copy(dacc.at[pl.ds(off, 640)], out_hbm.at[cid, pl.ds(off, 640)])


_deg_kernel = functools.partial(
    pl.kernel,
    out_type=jax.ShapeDtypeStruct((NC, NPAD), jnp.float32),
    mesh=_mesh,
    compiler_params=pltpu.CompilerParams(needs_layout_passes=False),
    scratch_types=[
        pltpu.VMEM((NCHUNK, K), jnp.int32),
        pltpu.VMEM((NCHUNK, K), jnp.float32),
        pltpu.VMEM((640,), jnp.float32),
        pltpu.VMEM_SHARED((NPAD,), jnp.float32),
    ],
)(_deg_body)


def _edge_body(m_hbm, src_hbm, dst_hbm, w_hbm, out_hbm,
               src2_v, dst2_v, w2_v, rows_a, rows_b, sem_a, sem_b, acc):
    cid = lax.axis_index("c")
    sid = lax.axis_index("s")
    wid = sid * NC + cid

    pltpu.sync_copy(src_hbm.at[wid], src2_v)
    pltpu.sync_copy(dst_hbm.at[wid], dst2_v)
    pltpu.sync_copy(w_hbm.at[wid], w2_v)

    def zero_row(r, carry):
        for t in range(8):
            rows_a[r, pl.ds(t * 16, 16)] = jnp.zeros((16,), jnp.float32)
        return carry
    lax.fori_loop(0, HK, zero_row, 0)

    row0 = pl.multiple_of(sid * RPT, 128)
    for q in range(RPT // HK):
        pltpu.sync_copy(rows_a, acc.at[pl.ds(row0 + q * HK, HK)])

    plsc.subcore_barrier()

    def _process(rows_v, c, off):
        def scale(j, inner):
            wj = plsc.load_gather(w2_v, [jnp.zeros((16,), jnp.int32) + c,
                                         jnp.zeros((16,), jnp.int32) + off + j])
            for t in range(8):
                sl = pl.ds(t * 16, 16)
                rows_v[j, sl] = rows_v[j, sl] * wj
            return inner
        lax.fori_loop(0, HK, scale, 0)
        pltpu.sync_copy(rows_v, acc.at[dst2_v.at[c, pl.ds(off, HK)]], add=True)

    def pair(c, carry):
        cp_a = pltpu.async_copy(
            m_hbm.at[src2_v.at[c, pl.ds(0, HK)]], rows_a, sem_a)
        cp_b = pltpu.async_copy(
            m_hbm.at[src2_v.at[c, pl.ds(HK, HK)]], rows_b, sem_b)
        cp_a.wait()
        _process(rows_a, c, 0)
        cp_b.wait()
        _process(rows_b, c, HK)
        return carry
    lax.fori_loop(0, NCHUNK, pair, 0)

    plsc.subcore_barrier()

    pltpu.sync_copy(acc.at[pl.ds(row0, RPT)],
                    out_hbm.at[cid, pl.ds(row0, RPT)])


_edge_kernel = functools.partial(
    pl.kernel,
    out_type=jax.ShapeDtypeStruct((NC, NPAD, DIN), jnp.float32),
    mesh=_mesh,
    compiler_params=pltpu.CompilerParams(needs_layout_passes=False),
    scratch_types=[
        pltpu.VMEM((NCHUNK, K), jnp.int32),
        pltpu.VMEM((NCHUNK, K), jnp.int32),
        pltpu.VMEM((NCHUNK, K), jnp.float32),
        pltpu.VMEM((HK, DIN), jnp.float32),
        pltpu.VMEM((HK, DIN), jnp.float32),
        pltpu.SemaphoreType.DMA,
        pltpu.SemaphoreType.DMA,
        pltpu.VMEM_SHARED((NPAD, DIN), jnp.float32),
    ],
)(_edge_body)


def _pre_body(d0_ref, d1_ref, x_ref, dinv_ref, xs_ref):
    d = d0_ref[0, 0, :, :] + d1_ref[0, 0, :, :] + 1.0
    dinv = lax.rsqrt(d)
    dinv_ref[0, :, :] = dinv
    xs_ref[...] = dinv * x_ref[...]


def _pre(degp, X):
    d4 = degp[:, :N].reshape(NC, NBLK, BLK, 1)
    return pl.pallas_call(
        _pre_body,
        grid=(NBLK,),
        in_specs=[
            pl.BlockSpec((1, 1, BLK, 1), lambda i: (0, i, 0, 0)),
            pl.BlockSpec((1, 1, BLK, 1), lambda i: (1, i, 0, 0)),
            pl.BlockSpec((BLK, DIN), lambda i: (i, 0)),
        ],
        out_specs=[
            pl.BlockSpec((1, BLK, 1), lambda i: (i, 0, 0)),
            pl.BlockSpec((BLK, DIN), lambda i: (i, 0)),
        ],
        out_shape=[
            jax.ShapeDtypeStruct((NBLK, BLK, 1), jnp.float32),
            jax.ShapeDtypeStruct((N, DIN), jnp.float32),
        ],
    )(d4, d4, X)


def _dense1_body(p0_ref, p1_ref, xs_ref, dinv_ref,
                 w1z_ref, b1z_ref, w2z_ref, w1h_ref, b1h_ref, w2h_ref,
                 us_ref):
    dinv = dinv_ref[0, :, :]
    y = (p0_ref[0, :, :] + p1_ref[0, :, :] + xs_ref[...]) * dinv
    t1z = jnp.maximum(jnp.dot(y, w1z_ref[...],
                              preferred_element_type=jnp.float32)
                      + b1z_ref[...], 0.0)
    t1h = jnp.maximum(jnp.dot(y, w1h_ref[...],
                              preferred_element_type=jnp.float32)
                      + b1h_ref[...], 0.0)
    uz = jnp.dot(t1z, w2z_ref[...], preferred_element_type=jnp.float32)
    uh = jnp.dot(t1h, w2h_ref[...], preferred_element_type=jnp.float32)
    us_ref[...] = jnp.concatenate([uz, uh], axis=1) * dinv


def _dense1(P1, dinv, Xs, W1_z, b1_z, W2_z, W1_h, b1_h, W2_h):
    whole = lambda shape: pl.BlockSpec(shape, lambda i: (0,) * len(shape))
    return pl.pallas_call(
        _dense1_body,
        grid=(NBLK,),
        in_specs=[
            pl.BlockSpec((1, BLK, DIN), lambda i: (0, i, 0)),
            pl.BlockSpec((1, BLK, DIN), lambda i: (1, i, 0)),
            pl.BlockSpec((BLK, DIN), lambda i: (i, 0)),
            pl.BlockSpec((1, BLK, 1), lambda i: (i, 0, 0)),
            whole((DIN, H0)), whole((1, H0)), whole((H0, H1)),
            whole((DIN, H0)), whole((1, H0)), whole((H0, H1)),
        ],
        out_specs=pl.BlockSpec((BLK, 2 * H1), lambda i: (i, 0)),
        out_shape=jax.ShapeDtypeStruct((N, 2 * H1), jnp.float32),
    )(P1, P1, Xs, dinv,
      W1_z, b1_z.reshape(1, H0), W2_z, W1_h, b1_h.reshape(1, H0), W2_h)


def _dense2_body(p0_ref, p1_ref, us_ref, dinv_ref, rb_ref, b2_ref, h_ref,
                 wlz_s_ref, wlz_h_ref, blz_ref, wlh_s_ref, wlh_h_ref, blh_ref,
                 out_ref, s_acc, c_acc):
    i = pl.program_id(0)

    @pl.when(i == 0)
    def _init():
        s_acc[...] = jnp.zeros_like(s_acc)
        c_acc[...] = jnp.zeros_like(c_acc)

    dinv = dinv_ref[0, :, :]
    v = (p0_ref[0, :, :] + p1_ref[0, :, :] + us_ref[...]) * dinv
    t2 = jnp.maximum(v + b2_ref[...], 0.0)
    rb = rb_ref[0, 0, :]
    gids = jax.lax.broadcasted_iota(jnp.int32, (G, BLK), 0)
    onehot = (gids == rb[None, :]).astype(jnp.float32)
    s_acc[...] += jnp.dot(onehot, t2, preferred_element_type=jnp.float32)
    c_acc[...] += jnp.broadcast_to(jnp.sum(onehot, axis=1, keepdims=True),
                                   (G, 2 * H1))

    @pl.when(i == NBLK - 1)
    def _epilogue():
        counts = jnp.maximum(c_acc[...], 1.0)
        s = s_acc[...] / counts
        sz = s[:, :H1]
        sh = s[:, H1:]
        h = h_ref[...]
        z = jax.nn.sigmoid(
            jnp.dot(sz, wlz_s_ref[...], preferred_element_type=jnp.float32)
            + jnp.dot(h, wlz_h_ref[...], preferred_element_type=jnp.float32)
            + blz_ref[...])
        ht = jnp.tanh(
            jnp.dot(sh, wlh_s_ref[...], preferred_element_type=jnp.float32)
            + jnp.dot(h, wlh_h_ref[...], preferred_element_type=jnp.float32)
            + blh_ref[...])
        out_ref[...] = z * h + (1.0 - z) * ht


def _dense2(P2, us, dinv, rb3, b2cat, H, Wl_z, bl_z, Wl_h, bl_h):
    whole = lambda shape: pl.BlockSpec(shape, lambda i: (0,) * len(shape))
    return pl.pallas_call(
        _dense2_body,
        grid=(NBLK,),
        in_specs=[
            pl.BlockSpec((1, BLK, 2 * H1), lambda i: (0, i, 0)),
            pl.BlockSpec((1, BLK, 2 * H1), lambda i: (1, i, 0)),
            pl.BlockSpec((BLK, 2 * H1), lambda i: (i, 0)),
            pl.BlockSpec((1, BLK, 1), lambda i: (i, 0, 0)),
            pl.BlockSpec((1, 1, BLK), lambda i: (i, 0, 0)),
            whole((1, 2 * H1)),
            whole((G, H1)),
            whole((H1, H1)), whole((H1, H1)), whole((1, H1)),
            whole((H1, H1)), whole((H1, H1)), whole((1, H1)),
        ],
        out_specs=whole((G, H1)),
        out_shape=jax.ShapeDtypeStruct((G, H1), jnp.float32),
        scratch_shapes=[
            pltpu.VMEM((G, 2 * H1), jnp.float32),
            pltpu.VMEM((G, 2 * H1), jnp.float32),
        ],
    )(P2, P2, us, dinv, rb3, b2cat.reshape(1, 2 * H1), H,
      Wl_z[:H1], Wl_z[H1:], bl_z.reshape(1, H1),
      Wl_h[:H1], Wl_h[H1:], bl_h.reshape(1, H1))


def kernel(X, edge_index, readout_batch, edge_weight, H,
           W1_z, b1_z, W2_z, b2_z, Wl_z, bl_z,
           W1_r, b1_r, W2_r, b2_r, Wl_r, bl_r,
           W1_h, b1_h, W2_h, b2_h, Wl_h, bl_h):
    pad = ((0, 0), (0, EWP - EW))
    src3 = jnp.pad(edge_index[0].reshape(NW, EW), pad).reshape(NW, NCHUNK, K)
    dst3 = jnp.pad(edge_index[1].reshape(NW, EW), pad).reshape(NW, NCHUNK, K)
    w3 = jnp.pad(edge_weight.reshape(NW, EW), pad).reshape(NW, NCHUNK, K)

    degp = _deg_kernel(dst3, w3)

    dinv, Xs = _pre(degp, X)

    P1 = _edge_kernel(Xs, src3, dst3, w3)

    us = _dense1(P1, dinv, Xs, W1_z, b1_z, W2_z, W1_h, b1_h, W2_h)

    P2 = _edge_kernel(us, src3, dst3, w3)

    rb3 = readout_batch.reshape(NBLK, 1, BLK)
    b2cat = jnp.concatenate([b2_z, b2_h])
    return _dense2(P2, us, dinv, rb3, b2cat, H, Wl_z, bl_z, Wl_h, bl_h)


# speedup vs baseline: 1.0831x; 1.0831x over previous
import functools

import jax
import jax.numpy as jnp
from jax import lax
from jax.experimental import pallas as pl
from jax.experimental.pallas import tpu as pltpu
from jax.experimental.pallas import tpu_sc as plsc

N = 10000
E = 320000
DIN = 128
H0 = 128
H1 = 64
G = 256

BLK = 400
NBLK = N // BLK

NC = 2
NS = 16
NW = NC * NS
EW = E // NW
K = 128
EWP = 10240
NCHUNK = EWP // K
HK = K // 2
RPT = 640
ZROWS = 32

_mesh = plsc.VectorSubcoreMesh(core_axis_name="c", subcore_axis_name="s")


NPAD = 10240


def _deg_body(dst_hbm, w_hbm, out_hbm, dst2_v, w2_v, zb_v, dacc):
    cid = lax.axis_index("c")
    sid = lax.axis_index("s")
    wid = sid * NC + cid

    pltpu.sync_copy(dst_hbm.at[wid], dst2_v)
    pltpu.sync_copy(w_hbm.at[wid], w2_v)

    def zero_row(r, carry):
        zb_v[pl.ds(r * 16, 16)] = jnp.zeros((16,), jnp.float32)
        return carry
    lax.fori_loop(0, 40, zero_row, 0)

    off = pl.multiple_of(sid * 640, 128)
    pltpu.sync_copy(zb_v, dacc.at[pl.ds(off, 640)])

    plsc.subcore_barrier()

    def chunk(c, carry):
        pltpu.sync_copy(w2_v.at[c], dacc.at[dst2_v.at[c]], add=True)
        return carry
    lax.fori_loop(0, NCHUNK, chunk, 0)

    plsc.subcore_barrier()

    pltpu.sync_copy(dacc.at[pl.ds(off, 640)], out_hbm.at[cid, pl.ds(off, 640)])


_deg_kernel = functools.partial(
    pl.kernel,
    out_type=jax.ShapeDtypeStruct((NC, NPAD), jnp.float32),
    mesh=_mesh,
    compiler_params=pltpu.CompilerParams(needs_layout_passes=False),
    scratch_types=[
        pltpu.VMEM((NCHUNK, K), jnp.int32),
        pltpu.VMEM((NCHUNK, K), jnp.float32),
        pltpu.VMEM((640,), jnp.float32),
        pltpu.VMEM_SHARED((NPAD,), jnp.float32),
    ],
)(_deg_body)


def _edge_body(m_hbm, src_hbm, dst_hbm, w_hbm, out_hbm,
               src2_v, dst2_v, w2_v, rows_v, sem, acc):
    cid = lax.axis_index("c")
    sid = lax.axis_index("s")
    wid = sid * NC + cid

    pltpu.sync_copy(src_hbm.at[wid], src2_v)
    pltpu.sync_copy(dst_hbm.at[wid], dst2_v)
    pltpu.sync_copy(w_hbm.at[wid], w2_v)

    @plsc.parallel_loop(0, K)
    def _zero(r):
        for t in range(8):
            rows_v[r, pl.ds(t * 16, 16)] = jnp.zeros((16,), jnp.float32)

    row0 = pl.multiple_of(sid * RPT, 128)
    for q in range(RPT // K):
        pltpu.sync_copy(rows_v, acc.at[pl.ds(row0 + q * K, K)])

    plsc.subcore_barrier()

    def chunk(c, carry):
        pltpu.async_copy(m_hbm.at[src2_v.at[c]], rows_v, sem).wait()

        @plsc.parallel_loop(0, K, unroll=4)
        def _scale(j):
            wj = plsc.load_gather(w2_v, [jnp.zeros((16,), jnp.int32) + c,
                                         jnp.zeros((16,), jnp.int32) + j])
            for t in range(8):
                sl = pl.ds(t * 16, 16)
                rows_v[j, sl] = rows_v[j, sl] * wj

        pltpu.sync_copy(rows_v, acc.at[dst2_v.at[c]], add=True)
        return carry
    lax.fori_loop(0, NCHUNK, chunk, 0)

    plsc.subcore_barrier()

    pltpu.sync_copy(acc.at[pl.ds(row0, RPT)],
                    out_hbm.at[cid, pl.ds(row0, RPT)])


_edge_kernel = functools.partial(
    pl.kernel,
    out_type=jax.ShapeDtypeStruct((NC, NPAD, DIN), jnp.float32),
    mesh=_mesh,
    compiler_params=pltpu.CompilerParams(needs_layout_passes=False),
    scratch_types=[
        pltpu.VMEM((NCHUNK, K), jnp.int32),
        pltpu.VMEM((NCHUNK, K), jnp.int32),
        pltpu.VMEM((NCHUNK, K), jnp.float32),
        pltpu.VMEM((K, DIN), jnp.float32),
        pltpu.SemaphoreType.DMA,
        pltpu.VMEM_SHARED((NPAD, DIN), jnp.float32),
    ],
)(_edge_body)


def _pre_body(d0_ref, d1_ref, x_ref, dinv_ref, xs_ref):
    d = d0_ref[0, 0, :, :] + d1_ref[0, 0, :, :] + 1.0
    dinv = lax.rsqrt(d)
    dinv_ref[0, :, :] = dinv
    xs_ref[...] = dinv * x_ref[...]


def _pre(degp, X):
    d4 = degp[:, :N].reshape(NC, NBLK, BLK, 1)
    return pl.pallas_call(
        _pre_body,
        grid=(NBLK,),
        in_specs=[
            pl.BlockSpec((1, 1, BLK, 1), lambda i: (0, i, 0, 0)),
            pl.BlockSpec((1, 1, BLK, 1), lambda i: (1, i, 0, 0)),
            pl.BlockSpec((BLK, DIN), lambda i: (i, 0)),
        ],
        out_specs=[
            pl.BlockSpec((1, BLK, 1), lambda i: (i, 0, 0)),
            pl.BlockSpec((BLK, DIN), lambda i: (i, 0)),
        ],
        out_shape=[
            jax.ShapeDtypeStruct((NBLK, BLK, 1), jnp.float32),
            jax.ShapeDtypeStruct((N, DIN), jnp.float32),
        ],
    )(d4, d4, X)


def _dense1_body(p0_ref, p1_ref, xs_ref, dinv_ref,
                 w1z_ref, b1z_ref, w2z_ref, w1h_ref, b1h_ref, w2h_ref,
                 us_ref):
    dinv = dinv_ref[0, :, :]
    y = (p0_ref[0, :, :] + p1_ref[0, :, :] + xs_ref[...]) * dinv
    t1z = jnp.maximum(jnp.dot(y, w1z_ref[...],
                              preferred_element_type=jnp.float32)
                      + b1z_ref[...], 0.0)
    t1h = jnp.maximum(jnp.dot(y, w1h_ref[...],
                              preferred_element_type=jnp.float32)
                      + b1h_ref[...], 0.0)
    uz = jnp.dot(t1z, w2z_ref[...], preferred_element_type=jnp.float32)
    uh = jnp.dot(t1h, w2h_ref[...], preferred_element_type=jnp.float32)
    us_ref[...] = jnp.concatenate([uz, uh], axis=1) * dinv


def _dense1(P1, dinv, Xs, W1_z, b1_z, W2_z, W1_h, b1_h, W2_h):
    whole = lambda shape: pl.BlockSpec(shape, lambda i: (0,) * len(shape))
    return pl.pallas_call(
        _dense1_body,
        grid=(NBLK,),
        in_specs=[
            pl.BlockSpec((1, BLK, DIN), lambda i: (0, i, 0)),
            pl.BlockSpec((1, BLK, DIN), lambda i: (1, i, 0)),
            pl.BlockSpec((BLK, DIN), lambda i: (i, 0)),
            pl.BlockSpec((1, BLK, 1), lambda i: (i, 0, 0)),
            whole((DIN, H0)), whole((1, H0)), whole((H0, H1)),
            whole((DIN, H0)), whole((1, H0)), whole((H0, H1)),
        ],
        out_specs=pl.BlockSpec((BLK, 2 * H1), lambda i: (i, 0)),
        out_shape=jax.ShapeDtypeStruct((N, 2 * H1), jnp.float32),
    )(P1, P1, Xs, dinv,
      W1_z, b1_z.reshape(1, H0), W2_z, W1_h, b1_h.reshape(1, H0), W2_h)


def _dense2_body(p0_ref, p1_ref, us_ref, dinv_ref, rb_ref, b2_ref, h_ref,
                 wlz_s_ref, wlz_h_ref, blz_ref, wlh_s_ref, wlh_h_ref, blh_ref,
                 out_ref, s_acc, c_acc):
    i = pl.program_id(0)

    @pl.when(i == 0)
    def _init():
        s_acc[...] = jnp.zeros_like(s_acc)
        c_acc[...] = jnp.zeros_like(c_acc)

    dinv = dinv_ref[0, :, :]
    v = (p0_ref[0, :, :] + p1_ref[0, :, :] + us_ref[...]) * dinv
    t2 = jnp.maximum(v + b2_ref[...], 0.0)
    rb = rb_ref[0, 0, :]
    gids = jax.lax.broadcasted_iota(jnp.int32, (G, BLK), 0)
    onehot = (gids == rb[None, :]).astype(jnp.float32)
    s_acc[...] += jnp.dot(onehot, t2, preferred_element_type=jnp.float32)
    c_acc[...] += jnp.broadcast_to(jnp.sum(onehot, axis=1, keepdims=True),
                                   (G, 2 * H1))

    @pl.when(i == NBLK - 1)
    def _epilogue():
        counts = jnp.maximum(c_acc[...], 1.0)
        s = s_acc[...] / counts
        sz = s[:, :H1]
        sh = s[:, H1:]
        h = h_ref[...]
        z = jax.nn.sigmoid(
            jnp.dot(sz, wlz_s_ref[...], preferred_element_type=jnp.float32)
            + jnp.dot(h, wlz_h_ref[...], preferred_element_type=jnp.float32)
            + blz_ref[...])
        ht = jnp.tanh(
            jnp.dot(sh, wlh_s_ref[...], preferred_element_type=jnp.float32)
            + jnp.dot(h, wlh_h_ref[...], preferred_element_type=jnp.float32)
            + blh_ref[...])
        out_ref[...] = z * h + (1.0 - z) * ht


def _dense2(P2, us, dinv, rb3, b2cat, H, Wl_z, bl_z, Wl_h, bl_h):
    whole = lambda shape: pl.BlockSpec(shape, lambda i: (0,) * len(shape))
    return pl.pallas_call(
        _dense2_body,
        grid=(NBLK,),
        in_specs=[
            pl.BlockSpec((1, BLK, 2 * H1), lambda i: (0, i, 0)),
            pl.BlockSpec((1, BLK, 2 * H1), lambda i: (1, i, 0)),
            pl.BlockSpec((BLK, 2 * H1), lambda i: (i, 0)),
            pl.BlockSpec((1, BLK, 1), lambda i: (i, 0, 0)),
            pl.BlockSpec((1, 1, BLK), lambda i: (i, 0, 0)),
            whole((1, 2 * H1)),
            whole((G, H1)),
            whole((H1, H1)), whole((H1, H1)), whole((1, H1)),
            whole((H1, H1)), whole((H1, H1)), whole((1, H1)),
        ],
        out_specs=whole((G, H1)),
        out_shape=jax.ShapeDtypeStruct((G, H1), jnp.float32),
        scratch_shapes=[
            pltpu.VMEM((G, 2 * H1), jnp.float32),
            pltpu.VMEM((G, 2 * H1), jnp.float32),
        ],
    )(P2, P2, us, dinv, rb3, b2cat.reshape(1, 2 * H1), H,
      Wl_z[:H1], Wl_z[H1:], bl_z.reshape(1, H1),
      Wl_h[:H1], Wl_h[H1:], bl_h.reshape(1, H1))


def kernel(X, edge_index, readout_batch, edge_weight, H,
           W1_z, b1_z, W2_z, b2_z, Wl_z, bl_z,
           W1_r, b1_r, W2_r, b2_r, Wl_r, bl_r,
           W1_h, b1_h, W2_h, b2_h, Wl_h, bl_h):
    pad = ((0, 0), (0, EWP - EW))
    src3 = jnp.pad(edge_index[0].reshape(NW, EW), pad).reshape(NW, NCHUNK, K)
    dst3 = jnp.pad(edge_index[1].reshape(NW, EW), pad).reshape(NW, NCHUNK, K)
    w3 = jnp.pad(edge_weight.reshape(NW, EW), pad).reshape(NW, NCHUNK, K)

    degp = _deg_kernel(dst3, w3)

    dinv, Xs = _pre(degp, X)

    P1 = _edge_kernel(Xs, src3, dst3, w3)

    us = _dense1(P1, dinv, Xs, W1_z, b1_z, W2_z, W1_h, b1_h, W2_h)

    P2 = _edge_kernel(us, src3, dst3, w3)

    rb3 = readout_batch.reshape(NBLK, 1, BLK)
    b2cat = jnp.concatenate([b2_z, b2_h])
    return _dense2(P2, us, dinv, rb3, b2cat, H, Wl_z, bl_z, Wl_h, bl_h)


# speedup vs baseline: 1.0886x; 1.0051x over previous
import functools

import jax
import jax.numpy as jnp
from jax import lax
from jax.experimental import pallas as pl
from jax.experimental.pallas import tpu as pltpu
from jax.experimental.pallas import tpu_sc as plsc

N = 10000
E = 320000
DIN = 128
H0 = 128
H1 = 64
G = 256

BLK = 400
NBLK = N // BLK

NC = 2
NS = 16
NW = NC * NS
EW = E // NW
K = 128
EWP = 10240
NCHUNK = EWP // K
HK = K // 2
RPT = 640
ZROWS = 32

_mesh = plsc.VectorSubcoreMesh(core_axis_name="c", subcore_axis_name="s")


NPAD = 10240


def _deg_body(dst_hbm, w_hbm, out_hbm, dst2_v, w2_v, zb_v, sem, dacc):
    cid = lax.axis_index("c")
    sid = lax.axis_index("s")
    wid = sid * NC + cid

    pltpu.sync_copy(dst_hbm.at[wid], dst2_v)
    pltpu.sync_copy(w_hbm.at[wid], w2_v)

    def zero_row(r, carry):
        zb_v[pl.ds(r * 16, 16)] = jnp.zeros((16,), jnp.float32)
        return carry
    lax.fori_loop(0, 40, zero_row, 0)

    off = pl.multiple_of(sid * 640, 128)
    pltpu.sync_copy(zb_v, dacc.at[pl.ds(off, 640)])

    plsc.subcore_barrier()

    def chunk(c, carry):
        pltpu.async_copy(w2_v.at[c], dacc.at[dst2_v.at[c]], sem, add=True)
        return carry
    lax.fori_loop(0, NCHUNK, chunk, 0)

    def drain(c, carry):
        pltpu.make_async_copy(w2_v.at[c], dacc.at[dst2_v.at[c]], sem).wait()
        return carry
    lax.fori_loop(0, NCHUNK, drain, 0)

    plsc.subcore_barrier()

    pltpu.sync_copy(dacc.at[pl.ds(off, 640)], out_hbm.at[cid, pl.ds(off, 640)])


_deg_kernel = functools.partial(
    pl.kernel,
    out_type=jax.ShapeDtypeStruct((NC, NPAD), jnp.float32),
    mesh=_mesh,
    compiler_params=pltpu.CompilerParams(needs_layout_passes=False),
    scratch_types=[
        pltpu.VMEM((NCHUNK, K), jnp.int32),
        pltpu.VMEM((NCHUNK, K), jnp.float32),
        pltpu.VMEM((640,), jnp.float32),
        pltpu.SemaphoreType.DMA,
        pltpu.VMEM_SHARED((NPAD,), jnp.float32),
    ],
)(_deg_body)


def _edge_body(m_hbm, src_hbm, dst_hbm, w_hbm, out_hbm,
               src2_v, dst2_v, w2_v, rows_v, sem, acc):
    cid = lax.axis_index("c")
    sid = lax.axis_index("s")
    wid = sid * NC + cid

    pltpu.sync_copy(src_hbm.at[wid], src2_v)
    pltpu.sync_copy(dst_hbm.at[wid], dst2_v)
    pltpu.sync_copy(w_hbm.at[wid], w2_v)

    @plsc.parallel_loop(0, K)
    def _zero(r):
        for t in range(8):
            rows_v[r, pl.ds(t * 16, 16)] = jnp.zeros((16,), jnp.float32)

    row0 = pl.multiple_of(sid * RPT, 128)
    for q in range(RPT // K):
        pltpu.sync_copy(rows_v, acc.at[pl.ds(row0 + q * K, K)])

    plsc.subcore_barrier()

    def chunk(c, carry):
        pltpu.async_copy(m_hbm.at[src2_v.at[c]], rows_v, sem).wait()

        @plsc.parallel_loop(0, K, unroll=4)
        def _scale(j):
            wj = plsc.load_gather(w2_v, [jnp.zeros((16,), jnp.int32) + c,
                                         jnp.zeros((16,), jnp.int32) + j])
            for t in range(8):
                sl = pl.ds(t * 16, 16)
                rows_v[j, sl] = rows_v[j, sl] * wj

        pltpu.sync_copy(rows_v, acc.at[dst2_v.at[c]], add=True)
        return carry
    lax.fori_loop(0, NCHUNK, chunk, 0)

    plsc.subcore_barrier()

    pltpu.sync_copy(acc.at[pl.ds(row0, RPT)],
                    out_hbm.at[cid, pl.ds(row0, RPT)])


_edge_kernel = functools.partial(
    pl.kernel,
    out_type=jax.ShapeDtypeStruct((NC, NPAD, DIN), jnp.float32),
    mesh=_mesh,
    compiler_params=pltpu.CompilerParams(needs_layout_passes=False),
    scratch_types=[
        pltpu.VMEM((NCHUNK, K), jnp.int32),
        pltpu.VMEM((NCHUNK, K), jnp.int32),
        pltpu.VMEM((NCHUNK, K), jnp.float32),
        pltpu.VMEM((K, DIN), jnp.float32),
        pltpu.SemaphoreType.DMA,
        pltpu.VMEM_SHARED((NPAD, DIN), jnp.float32),
    ],
)(_edge_body)


def _pre_body(d0_ref, d1_ref, x_ref, dinv_ref, xs_ref):
    d = d0_ref[0, 0, :, :] + d1_ref[0, 0, :, :] + 1.0
    dinv = lax.rsqrt(d)
    dinv_ref[0, :, :] = dinv
    xs_ref[...] = dinv * x_ref[...]


def _pre(degp, X):
    d4 = degp[:, :N].reshape(NC, NBLK, BLK, 1)
    return pl.pallas_call(
        _pre_body,
        grid=(NBLK,),
        in_specs=[
            pl.BlockSpec((1, 1, BLK, 1), lambda i: (0, i, 0, 0)),
            pl.BlockSpec((1, 1, BLK, 1), lambda i: (1, i, 0, 0)),
            pl.BlockSpec((BLK, DIN), lambda i: (i, 0)),
        ],
        out_specs=[
            pl.BlockSpec((1, BLK, 1), lambda i: (i, 0, 0)),
            pl.BlockSpec((BLK, DIN), lambda i: (i, 0)),
        ],
        out_shape=[
            jax.ShapeDtypeStruct((NBLK, BLK, 1), jnp.float32),
            jax.ShapeDtypeStruct((N, DIN), jnp.float32),
        ],
    )(d4, d4, X)


def _dense1_body(p0_ref, p1_ref, xs_ref, dinv_ref,
                 w1z_ref, b1z_ref, w2z_ref, w1h_ref, b1h_ref, w2h_ref,
                 us_ref):
    dinv = dinv_ref[0, :, :]
    y = (p0_ref[0, :, :] + p1_ref[0, :, :] + xs_ref[...]) * dinv
    t1z = jnp.maximum(jnp.dot(y, w1z_ref[...],
                              preferred_element_type=jnp.float32)
                      + b1z_ref[...], 0.0)
    t1h = jnp.maximum(jnp.dot(y, w1h_ref[...],
                              preferred_element_type=jnp.float32)
                      + b1h_ref[...], 0.0)
    uz = jnp.dot(t1z, w2z_ref[...], preferred_element_type=jnp.float32)
    uh = jnp.dot(t1h, w2h_ref[...], preferred_element_type=jnp.float32)
    us_ref[...] = jnp.concatenate([uz, uh], axis=1) * dinv


def _dense1(P1, dinv, Xs, W1_z, b1_z, W2_z, W1_h, b1_h, W2_h):
    whole = lambda shape: pl.BlockSpec(shape, lambda i: (0,) * len(shape))
    return pl.pallas_call(
        _dense1_body,
        grid=(NBLK,),
        in_specs=[
            pl.BlockSpec((1, BLK, DIN), lambda i: (0, i, 0)),
            pl.BlockSpec((1, BLK, DIN), lambda i: (1, i, 0)),
            pl.BlockSpec((BLK, DIN), lambda i: (i, 0)),
            pl.BlockSpec((1, BLK, 1), lambda i: (i, 0, 0)),
            whole((DIN, H0)), whole((1, H0)), whole((H0, H1)),
            whole((DIN, H0)), whole((1, H0)), whole((H0, H1)),
        ],
        out_specs=pl.BlockSpec((BLK, 2 * H1), lambda i: (i, 0)),
        out_shape=jax.ShapeDtypeStruct((N, 2 * H1), jnp.float32),
    )(P1, P1, Xs, dinv,
      W1_z, b1_z.reshape(1, H0), W2_z, W1_h, b1_h.reshape(1, H0), W2_h)


def _dense2_body(p0_ref, p1_ref, us_ref, dinv_ref, rb_ref, b2_ref, h_ref,
                 wlz_s_ref, wlz_h_ref, blz_ref, wlh_s_ref, wlh_h_ref, blh_ref,
                 out_ref, s_acc, c_acc):
    i = pl.program_id(0)

    @pl.when(i == 0)
    def _init():
        s_acc[...] = jnp.zeros_like(s_acc)
        c_acc[...] = jnp.zeros_like(c_acc)

    dinv = dinv_ref[0, :, :]
    v = (p0_ref[0, :, :] + p1_ref[0, :, :] + us_ref[...]) * dinv
    t2 = jnp.maximum(v + b2_ref[...], 0.0)
    rb = rb_ref[0, 0, :]
    gids = jax.lax.broadcasted_iota(jnp.int32, (G, BLK), 0)
    onehot = (gids == rb[None, :]).astype(jnp.float32)
    s_acc[...] += jnp.dot(onehot, t2, preferred_element_type=jnp.float32)
    c_acc[...] += jnp.broadcast_to(jnp.sum(onehot, axis=1, keepdims=True),
                                   (G, 2 * H1))

    @pl.when(i == NBLK - 1)
    def _epilogue():
        counts = jnp.maximum(c_acc[...], 1.0)
        s = s_acc[...] / counts
        sz = s[:, :H1]
        sh = s[:, H1:]
        h = h_ref[...]
        z = jax.nn.sigmoid(
            jnp.dot(sz, wlz_s_ref[...], preferred_element_type=jnp.float32)
            + jnp.dot(h, wlz_h_ref[...], preferred_element_type=jnp.float32)
            + blz_ref[...])
        ht = jnp.tanh(
            jnp.dot(sh, wlh_s_ref[...], preferred_element_type=jnp.float32)
            + jnp.dot(h, wlh_h_ref[...], preferred_element_type=jnp.float32)
            + blh_ref[...])
        out_ref[...] = z * h + (1.0 - z) * ht


def _dense2(P2, us, dinv, rb3, b2cat, H, Wl_z, bl_z, Wl_h, bl_h):
    whole = lambda shape: pl.BlockSpec(shape, lambda i: (0,) * len(shape))
    return pl.pallas_call(
        _dense2_body,
        grid=(NBLK,),
        in_specs=[
            pl.BlockSpec((1, BLK, 2 * H1), lambda i: (0, i, 0)),
            pl.BlockSpec((1, BLK, 2 * H1), lambda i: (1, i, 0)),
            pl.BlockSpec((BLK, 2 * H1), lambda i: (i, 0)),
            pl.BlockSpec((1, BLK, 1), lambda i: (i, 0, 0)),
            pl.BlockSpec((1, 1, BLK), lambda i: (i, 0, 0)),
            whole((1, 2 * H1)),
            whole((G, H1)),
            whole((H1, H1)), whole((H1, H1)), whole((1, H1)),
            whole((H1, H1)), whole((H1, H1)), whole((1, H1)),
        ],
        out_specs=whole((G, H1)),
        out_shape=jax.ShapeDtypeStruct((G, H1), jnp.float32),
        scratch_shapes=[
            pltpu.VMEM((G, 2 * H1), jnp.float32),
            pltpu.VMEM((G, 2 * H1), jnp.float32),
        ],
    )(P2, P2, us, dinv, rb3, b2cat.reshape(1, 2 * H1), H,
      Wl_z[:H1], Wl_z[H1:], bl_z.reshape(1, H1),
      Wl_h[:H1], Wl_h[H1:], bl_h.reshape(1, H1))


def kernel(X, edge_index, readout_batch, edge_weight, H,
           W1_z, b1_z, W2_z, b2_z, Wl_z, bl_z,
           W1_r, b1_r, W2_r, b2_r, Wl_r, bl_r,
           W1_h, b1_h, W2_h, b2_h, Wl_h, bl_h):
    pad = ((0, 0), (0, EWP - EW))
    src3 = jnp.pad(edge_index[0].reshape(NW, EW), pad).reshape(NW, NCHUNK, K)
    dst3 = jnp.pad(edge_index[1].reshape(NW, EW), pad).reshape(NW, NCHUNK, K)
    w3 = jnp.pad(edge_weight.reshape(NW, EW), pad).reshape(NW, NCHUNK, K)

    degp = _deg_kernel(dst3, w3)

    dinv, Xs = _pre(degp, X)

    P1 = _edge_kernel(Xs, src3, dst3, w3)

    us = _dense1(P1, dinv, Xs, W1_z, b1_z, W2_z, W1_h, b1_h, W2_h)

    P2 = _edge_kernel(us, src3, dst3, w3)

    rb3 = readout_batch.reshape(NBLK, 1, BLK)
    b2cat = jnp.concatenate([b2_z, b2_h])
    return _dense2(P2, us, dinv, rb3, b2cat, H, Wl_z, bl_z, Wl_h, bl_h)
